# baseline (speedup 1.0000x reference)
import functools
import math

import jax
import jax.numpy as jnp
from jax import lax
from jax.experimental import pallas as pl
from jax.experimental.pallas import tpu as pltpu
from jax.experimental.pallas import tpu_sc as plsc

T = 4096
D = 768
E = 64
FF = 3072
CAP = 64
NROWS = E * CAP + CAP
TRASH = E * CAP
NTR = 34

NC = 2
NS = 16
NW = NC * NS
TPW = T // NW


def _gelu(x):
    c = math.sqrt(2.0 / math.pi)
    return x * 0.5 * (1.0 + jnp.tanh(c * (x + 0.044715 * x * x * x)))


def _router_body(x_ref, wr_ref, disp_ref, tok_ref, gates_ref, oh_ref, p_ref):
    x = x_ref[...]
    logits = jnp.dot(x, wr_ref[...], preferred_element_type=jnp.float32)
    m = jnp.max(logits, axis=1, keepdims=True)
    gate = 1.0 / jnp.sum(jnp.exp(logits - m), axis=1, keepdims=True)
    lane = lax.broadcasted_iota(jnp.int32, (T, E), 1).astype(jnp.float32)
    cand = jnp.where(logits == m, lane, 1e9)
    e_f = jnp.min(cand, axis=1, keepdims=True)
    onehot = (lane == e_f).astype(jnp.float32)
    oh_ref[...] = onehot

    CH = 128
    r = lax.broadcasted_iota(jnp.int32, (CH, CH), 0)
    c = lax.broadcasted_iota(jnp.int32, (CH, CH), 1)
    tri = (r >= c).astype(jnp.float32)

    def body(i, carry):
        mc = oh_ref[pl.ds(i * CH, CH), :]
        incl = jnp.dot(tri, mc, preferred_element_type=jnp.float32) + carry
        p_ref[pl.ds(i * CH, CH), :] = jnp.sum(incl * mc, axis=1, keepdims=True)
        return carry + jnp.sum(mc, axis=0, keepdims=True)

    lax.fori_loop(0, T // CH, body, jnp.zeros((1, E), jnp.float32))

    p = p_ref[...]
    keep = p < float(CAP)
    slot = e_f.astype(jnp.int32) * CAP + p.astype(jnp.int32) - 1
    disp_ref[...] = jnp.where(keep, slot, TRASH)

    cslot = jnp.where(keep, p.astype(jnp.int32) - 1, 2 * CAP)
    pos1h = (lax.broadcasted_iota(jnp.int32, (T, CAP), 1) == cslot)
    pos1h = pos1h.astype(jnp.float32)
    tiota = lax.broadcasted_iota(jnp.int32, (T, 1), 0)
    tq = lax.shift_right_logical(tiota, 6).astype(jnp.float32)
    tr = lax.bitwise_and(tiota, 63).astype(jnp.float32)
    dn = (((0,), (0,)), ((), ()))
    hi = lax.Precision.HIGHEST
    tokq = lax.dot_general(onehot * tq, pos1h, dn,
                           preferred_element_type=jnp.float32)
    tokr = lax.dot_general(onehot * tr, pos1h, dn,
                           preferred_element_type=jnp.float32)
    cnt = lax.dot_general(onehot, pos1h, dn,
                          preferred_element_type=jnp.float32)
    gat = lax.dot_general(onehot * gate, pos1h, dn, precision=hi,
                          preferred_element_type=jnp.float32)
    tok = tokq.astype(jnp.int32) * 64 + tokr.astype(jnp.int32)
    tok_ref[...] = jnp.where(cnt > 0.0, tok, T)
    gates_ref[...] = gat


def _router(x, Wr):
    return pl.pallas_call(
        _router_body,
        out_shape=[
            jax.ShapeDtypeStruct((T, 1), jnp.int32),
            jax.ShapeDtypeStruct((E, CAP), jnp.int32),
            jax.ShapeDtypeStruct((E, CAP), jnp.float32),
        ],
        scratch_shapes=[
            pltpu.VMEM((T, E), jnp.float32),
            pltpu.VMEM((T, 1), jnp.float32),
        ],
    )(x, Wr)


@functools.lru_cache(maxsize=None)
def _make_dispatch():
    mesh = plsc.VectorSubcoreMesh(core_axis_name="c", subcore_axis_name="s")

    @functools.partial(
        pl.kernel,
        out_type=jax.ShapeDtypeStruct((NROWS, D), jnp.float32),
        mesh=mesh,
        scratch_types=[
            pltpu.VMEM((TPW,), jnp.int32),
            pltpu.VMEM((TPW, D), jnp.float32),
            pltpu.SemaphoreType.DMA,
        ],
        compiler_params=pltpu.CompilerParams(needs_layout_passes=False),
    )
    def _dispatch(x_hbm, idx_hbm, ei_hbm, idx_v, rows_v, sem):
        wid = lax.axis_index("s") * NC + lax.axis_index("c")
        base = wid * TPW
        pltpu.sync_copy(idx_hbm.at[pl.ds(base, TPW)], idx_v)
        pltpu.sync_copy(x_hbm.at[pl.ds(base, TPW)], rows_v)
        pltpu.async_copy(rows_v, ei_hbm.at[idx_v], sem).wait()

    return _dispatch


def _mlp_body(tok_ref, ei_ref, w1_ref, b1_ref, w2_ref, b2_ref, gates_ref,
              out_ref, eo_s):
    e = pl.program_id(0)

    @pl.when(e == 0)
    def _zero():
        out_ref[...] = jnp.zeros_like(out_ref)

    ei = ei_ref[...].astype(jnp.bfloat16)
    h = jnp.dot(ei, w1_ref[0].astype(jnp.bfloat16),
                preferred_element_type=jnp.float32)
    h = _gelu(h + b1_ref[e]).astype(jnp.bfloat16)
    o = jnp.dot(h, w2_ref[0].astype(jnp.bfloat16),
                preferred_element_type=jnp.float32)
    eo_s[...] = _gelu(o + b2_ref[e]) * gates_ref[e]

    def row(rI, carry):
        t = tok_ref[e * CAP + rI]

        @pl.when(t < T)
        def _store():
            out_ref[pl.ds(t, 1), :] = eo_s[pl.ds(rI, 1), :]

        return carry

    lax.fori_loop(0, CAP, row, 0)


def _mlp(tok, ei, W1, b1, W2, b2, gates):
    grid_spec = pltpu.PrefetchScalarGridSpec(
        num_scalar_prefetch=1,
        grid=(E,),
        in_specs=[
            pl.BlockSpec((CAP, D), lambda e, tok: (e, 0)),
            pl.BlockSpec((1, D, FF), lambda e, tok: (e, 0, 0)),
            pl.BlockSpec((E, 1, FF), lambda e, tok: (0, 0, 0)),
            pl.BlockSpec((1, FF, D), lambda e, tok: (e, 0, 0)),
            pl.BlockSpec((E, 1, D), lambda e, tok: (0, 0, 0)),
            pl.BlockSpec((E, CAP, 1), lambda e, tok: (0, 0, 0)),
        ],
        out_specs=pl.BlockSpec((T, D), lambda e, tok: (0, 0)),
        scratch_shapes=[pltpu.VMEM((CAP, D), jnp.float32)],
    )
    return pl.pallas_call(
        _mlp_body,
        grid_spec=grid_spec,
        out_shape=jax.ShapeDtypeStruct((T, D), jnp.float32),
    )(tok, ei, W1, b1.reshape(E, 1, FF), W2, b2.reshape(E, 1, D),
      gates.reshape(E, CAP, 1))


def kernel(inputs, Wr, W1, b1, W2, b2):
    x = inputs.reshape(T, D)
    disp_idx, tok, gates = _router(x, Wr)
    ei = _make_dispatch()(x, disp_idx.reshape(T))
    out = _mlp(tok.reshape(E * CAP), ei, W1, b1, W2, b2, gates)
    return out.reshape(inputs.shape)
